# baseline (speedup 1.0000x reference)
import functools

import jax
import jax.numpy as jnp
from jax import lax
from jax.experimental import pallas as pl
from jax.experimental.pallas import tpu as pltpu
from jax.experimental.pallas import tpu_sc as plsc

N, D, E = 2048, 768, 64
NSLOT = 2 * N
TM = 256
NT = NSLOT // TM
NSTEPS = NT + E - 1
BLK = 256
NBLK = N // BLK
NW = 32
TPW = N // NW
SPW = NSLOT // NW
EPS = 2.220446049250313e-16
NEG_INF = float("-inf")


def _router_kernel(flat_ref, wg_ref, coef_ref,
                   pos0_ref, pos1_ref, g1_ref, g2_ref, steps_ref, loss_ref,
                   flat_lin_ref, e1_s, e2_s):
    flat = flat_ref[...]
    flat_lin_ref[...] = flat
    logits = jnp.dot(flat, wg_ref[...], preferred_element_type=jnp.float32)
    lane = lax.broadcasted_iota(jnp.int32, (N, E), 1)
    m1 = jnp.max(logits, axis=1, keepdims=True)
    e1 = jnp.min(jnp.where(logits == m1, lane, E), axis=1, keepdims=True)
    masked = jnp.where(lane == e1, NEG_INF, logits)
    m2 = jnp.max(masked, axis=1, keepdims=True)
    e2 = jnp.min(jnp.where(masked == m2, lane, E), axis=1, keepdims=True)
    z2 = jnp.exp(m2 - m1)
    g1 = 1.0 / (1.0 + z2)
    g2 = z2 / (1.0 + z2)
    g1_ref[...] = g1
    g2_ref[...] = g2
    e1_s[...] = e1
    e2_s[...] = e2

    onehot1 = (lane == e1).astype(jnp.float32)
    onehot2 = (lane == e2).astype(jnp.float32)
    count1 = jnp.sum(onehot1, axis=0, keepdims=True)
    count2 = jnp.sum(onehot2, axis=0, keepdims=True)
    counts = count1 + count2

    gates = jnp.where(lane == e1, g1, 0.0) + jnp.where(lane == e2, g2, 0.0)
    importance = jnp.sum(gates, axis=0, keepdims=True)
    load = jnp.sum((gates > 0.0).astype(jnp.float32), axis=0, keepdims=True)

    def cv2(v):
        m = jnp.sum(v) / E
        var = jnp.sum((v - m) ** 2) / (E - 1)
        return var / (m * m + 1e-10)

    loss_ref[0, 0] = (cv2(importance) + cv2(load)) * coef_ref[0, 0]

    r64 = lax.broadcasted_iota(jnp.int32, (E, E), 0)
    c64 = lax.broadcasted_iota(jnp.int32, (E, E), 1)
    excl64 = (r64 < c64).astype(jnp.float32)
    offs = jnp.dot(counts, excl64, preferred_element_type=jnp.float32)
    base1 = offs
    base2 = offs + count1

    incl64 = (r64 <= c64).astype(jnp.float32)
    off_incl = jnp.dot(counts, incl64, preferred_element_type=jnp.float32)
    off_excl = offs
    cnt_pos = counts > 0.0
    ft = jnp.floor(off_excl * (1.0 / TM))
    lt = jnp.where(cnt_pos, jnp.floor((off_incl - 1.0) * (1.0 / TM)), ft)
    items = jnp.where(cnt_pos, lt - ft + 1.0, 0.0)
    sitem_excl = jnp.dot(items, excl64, preferred_element_type=jnp.float32)
    sitem_incl = sitem_excl + items
    total = jnp.sum(items)
    jcol = lax.broadcasted_iota(
        jnp.int32, (NSTEPS, 1), 0).astype(jnp.float32)
    e_of = jnp.sum((sitem_incl <= jcol).astype(jnp.float32),
                   axis=1, keepdims=True)
    e_ofc = jnp.minimum(e_of, float(E - 1))
    lane_s = lax.broadcasted_iota(jnp.int32, (NSTEPS, E), 1)
    onehot_e = (lane_s == e_ofc.astype(jnp.int32)).astype(jnp.float32)
    ft_j = jnp.sum(onehot_e * ft, axis=1, keepdims=True)
    se_j = jnp.sum(onehot_e * sitem_excl, axis=1, keepdims=True)
    gs_j = jnp.sum(onehot_e * off_excl, axis=1, keepdims=True)
    ge_j = jnp.sum(onehot_e * off_incl, axis=1, keepdims=True)
    valid = jcol < total
    t_of = jnp.where(valid, ft_j + (jcol - se_j), float(NT - 1))
    e_pad = jnp.sum(jnp.where(jcol == total - 1.0, e_ofc, 0.0))
    e_fin = jnp.where(valid, e_ofc, e_pad)
    gs_f = jnp.where(valid, gs_j, 0.0)
    ge_f = jnp.where(valid, ge_j, 0.0)
    row0 = t_of * TM
    first = valid & (gs_f <= row0) & (row0 < ge_f)
    steps = jnp.concatenate(
        [t_of, e_fin, gs_f, ge_f,
         valid.astype(jnp.float32), first.astype(jnp.float32),
         jnp.zeros((NSTEPS, 2), jnp.float32)], axis=1)
    steps_ref[...] = steps.astype(jnp.int32)

    rblk = lax.broadcasted_iota(jnp.int32, (BLK, BLK), 0)
    cblk = lax.broadcasted_iota(jnp.int32, (BLK, BLK), 1)
    tri = (rblk > cblk).astype(jnp.float32)
    lane_b = lax.broadcasted_iota(jnp.int32, (BLK, E), 1)

    def body(b, carry):
        run1, run2 = carry
        e1b = e1_s[pl.ds(b * BLK, BLK), :]
        e2b = e2_s[pl.ds(b * BLK, BLK), :]
        oh1 = (lane_b == e1b).astype(jnp.float32)
        oh2 = (lane_b == e2b).astype(jnp.float32)
        pref1 = jnp.dot(tri, oh1, preferred_element_type=jnp.float32) + run1
        pref2 = jnp.dot(tri, oh2, preferred_element_type=jnp.float32) + run2
        p0 = (jnp.sum(pref1 * oh1, axis=1, keepdims=True)
              + jnp.sum(oh1 * base1, axis=1, keepdims=True))
        p1 = (jnp.sum(pref2 * oh2, axis=1, keepdims=True)
              + jnp.sum(oh2 * base2, axis=1, keepdims=True))
        pos0_ref[pl.ds(b * BLK, BLK), :] = p0.astype(jnp.int32)
        pos1_ref[pl.ds(b * BLK, BLK), :] = p1.astype(jnp.int32)
        return (run1 + jnp.sum(oh1, axis=0, keepdims=True),
                run2 + jnp.sum(oh2, axis=0, keepdims=True))

    lax.fori_loop(0, NBLK, body,
                  (jnp.zeros((1, E), jnp.float32),
                   jnp.zeros((1, E), jnp.float32)))


def _router_call(flat, coef, w_gate):
    return pl.pallas_call(
        _router_kernel,
        in_specs=[
            pl.BlockSpec((N, D), lambda: (0, 0)),
            pl.BlockSpec((D, E), lambda: (0, 0)),
            pl.BlockSpec(memory_space=pltpu.SMEM),
        ],
        out_specs=[
            pl.BlockSpec((N, 1), lambda: (0, 0)),
            pl.BlockSpec((N, 1), lambda: (0, 0)),
            pl.BlockSpec((N, 1), lambda: (0, 0)),
            pl.BlockSpec((N, 1), lambda: (0, 0)),
            pl.BlockSpec((NSTEPS, 8), lambda: (0, 0)),
            pl.BlockSpec(memory_space=pltpu.SMEM),
            pl.BlockSpec((N, D), lambda: (0, 0)),
        ],
        out_shape=[
            jax.ShapeDtypeStruct((N, 1), jnp.int32),
            jax.ShapeDtypeStruct((N, 1), jnp.int32),
            jax.ShapeDtypeStruct((N, 1), jnp.float32),
            jax.ShapeDtypeStruct((N, 1), jnp.float32),
            jax.ShapeDtypeStruct((NSTEPS, 8), jnp.int32),
            jax.ShapeDtypeStruct((1, 1), jnp.float32),
            jax.ShapeDtypeStruct((N, D), jnp.float32),
        ],
        scratch_shapes=[
            pltpu.VMEM((N, 1), jnp.int32),
            pltpu.VMEM((N, 1), jnp.int32),
        ],
    )(flat, w_gate, coef)


def _gmm_kernel(steps_ref, xs_ref, W_ref, b_ref, out_ref):
    s = pl.program_id(0)
    valid = steps_ref[s, 4]

    @pl.when(valid == 1)
    def _():
        t = steps_ref[s, 0]
        g_start = steps_ref[s, 2]
        g_end = steps_ref[s, 3]
        first = steps_ref[s, 5]
        rows = t * TM + lax.broadcasted_iota(jnp.int32, (TM, 1), 0)
        in_seg = (rows >= g_start) & (rows < g_end)
        out = jnp.dot(xs_ref[...], W_ref[0],
                      preferred_element_type=jnp.float32) + b_ref[0]
        expo = jnp.exp(out)
        prev = jnp.where(first == 1, jnp.zeros_like(expo), out_ref[...])
        out_ref[...] = jnp.where(in_seg, expo, prev)


def _gmm_call(steps, xs, expert_W, expert_b):
    grid_spec = pltpu.PrefetchScalarGridSpec(
        num_scalar_prefetch=1,
        grid=(NSTEPS,),
        in_specs=[
            pl.BlockSpec((TM, D), lambda s, st: (st[s, 0], 0)),
            pl.BlockSpec((1, D, D), lambda s, st: (st[s, 1], 0, 0)),
            pl.BlockSpec((1, 1, D), lambda s, st: (st[s, 1], 0, 0)),
        ],
        out_specs=pl.BlockSpec((TM, D), lambda s, st: (st[s, 0], 0)),
    )
    return pl.pallas_call(
        _gmm_kernel,
        grid_spec=grid_spec,
        out_shape=jax.ShapeDtypeStruct((NSLOT, D), jnp.float32),
        compiler_params=pltpu.CompilerParams(
            dimension_semantics=("arbitrary",)),
    )(steps, xs, expert_W, expert_b.reshape(E, 1, D))


@functools.cache
def _sc_kernels():
    mesh = plsc.VectorSubcoreMesh(core_axis_name="c", subcore_axis_name="s")

    @functools.partial(
        pl.kernel,
        mesh=mesh,
        out_type=jax.ShapeDtypeStruct((NSLOT, D), jnp.float32),
        scratch_types=[
            pltpu.VMEM((TPW,), jnp.int32),
            pltpu.VMEM((TPW,), jnp.int32),
            pltpu.VMEM((TPW, D), jnp.float32),
            pltpu.SemaphoreType.DMA,
            pltpu.SemaphoreType.DMA,
        ],
    )
    def dispatch_sc(flat_hbm, pos0_hbm, pos1_hbm, xs_hbm,
                    idx0_v, idx1_v, rows_v, sem0, sem1):
        wid = lax.axis_index("s") * 2 + lax.axis_index("c")
        base = wid * TPW
        pltpu.sync_copy(pos0_hbm.at[pl.ds(base, TPW)], idx0_v)
        pltpu.sync_copy(pos1_hbm.at[pl.ds(base, TPW)], idx1_v)
        pltpu.sync_copy(flat_hbm.at[pl.ds(base, TPW)], rows_v)
        c0 = pltpu.async_copy(rows_v, xs_hbm.at[idx0_v], sem0)
        c1 = pltpu.async_copy(rows_v, xs_hbm.at[idx1_v], sem1)
        c0.wait()
        c1.wait()

    @functools.partial(
        pl.kernel,
        mesh=mesh,
        out_type=[
            jax.ShapeDtypeStruct((N, D), jnp.float32),
            jax.ShapeDtypeStruct((N, D), jnp.float32),
        ],
        scratch_types=[
            pltpu.VMEM((TPW,), jnp.int32),
            pltpu.VMEM((TPW,), jnp.int32),
            pltpu.VMEM((TPW, D), jnp.float32),
            pltpu.VMEM((TPW, D), jnp.float32),
            pltpu.SemaphoreType.DMA,
            pltpu.SemaphoreType.DMA,
        ],
    )
    def combine_sc(expo_hbm, pos0_hbm, pos1_hbm, c0_hbm, c1_hbm,
                   idx0_v, idx1_v, rows0_v, rows1_v, sem0, sem1):
        wid = lax.axis_index("s") * 2 + lax.axis_index("c")
        base = wid * TPW
        pltpu.sync_copy(pos0_hbm.at[pl.ds(base, TPW)], idx0_v)
        pltpu.sync_copy(pos1_hbm.at[pl.ds(base, TPW)], idx1_v)
        g0 = pltpu.async_copy(expo_hbm.at[idx0_v], rows0_v, sem0)
        g1 = pltpu.async_copy(expo_hbm.at[idx1_v], rows1_v, sem1)
        g0.wait()
        g1.wait()
        pltpu.sync_copy(rows0_v, c0_hbm.at[pl.ds(base, TPW)])
        pltpu.sync_copy(rows1_v, c1_hbm.at[pl.ds(base, TPW)])

    return dispatch_sc, combine_sc


def _final_kernel(c0_ref, c1_ref, g1_ref, g2_ref, y_ref):
    s = g1_ref[...] * c0_ref[...] + g2_ref[...] * c1_ref[...]
    y_ref[...] = jnp.log(jnp.where(s == 0.0, EPS, s))


def _final_call(c0, c1, g1, g2):
    nb = 8
    blk = N // nb
    return pl.pallas_call(
        _final_kernel,
        grid=(nb,),
        in_specs=[
            pl.BlockSpec((blk, D), lambda i: (i, 0)),
            pl.BlockSpec((blk, D), lambda i: (i, 0)),
            pl.BlockSpec((blk, 1), lambda i: (i, 0)),
            pl.BlockSpec((blk, 1), lambda i: (i, 0)),
        ],
        out_specs=pl.BlockSpec((blk, D), lambda i: (i, 0)),
        out_shape=jax.ShapeDtypeStruct((N, D), jnp.float32),
    )(c0, c1, g1, g2)


def kernel(x, loss_coef, w_gate, expert_W, expert_b):
    flat = x.reshape(N, D)
    coef = loss_coef.reshape(1, 1)
    pos0, pos1, g1, g2, steps, loss, flat_lin = _router_call(
        flat, coef, w_gate)
    p0 = pos0.reshape(N)
    p1 = pos1.reshape(N)
    dispatch_sc, combine_sc = _sc_kernels()
    xs = dispatch_sc(flat_lin, p0, p1)
    expo = _gmm_call(steps, xs, expert_W, expert_b)
    c0, c1 = combine_sc(expo, p0, p1)
    y = _final_call(c0, c1, g1, g2)
    return y, loss[0, 0]


# speedup vs baseline: 1.0402x; 1.0402x over previous
import functools

import jax
import jax.numpy as jnp
from jax import lax
from jax.experimental import pallas as pl
from jax.experimental.pallas import tpu as pltpu
from jax.experimental.pallas import tpu_sc as plsc

N, D, E = 2048, 768, 64
NSLOT = 2 * N
TM = 256
NT = NSLOT // TM
NSTEPS = NT + E - 1
BLK = 256
NBLK = N // BLK
NW = 32
TPW = N // NW
SPW = NSLOT // NW
EPS = 2.220446049250313e-16
NEG_INF = float("-inf")


def _router_kernel(x_hbm, wg_ref, coef_ref,
                   pos0_ref, pos1_ref, g1_ref, g2_ref, steps_ref, loss_ref,
                   flat_lin_ref, e1_s, e2_s, flat_s, dma_sem):
    cp = pltpu.make_async_copy(x_hbm.at[:, 0, :], flat_s, dma_sem)
    cp.start()
    cp.wait()
    flat = flat_s[...]
    flat_lin_ref[...] = flat
    logits = jnp.dot(flat, wg_ref[...], preferred_element_type=jnp.float32)
    lane = lax.broadcasted_iota(jnp.int32, (N, E), 1)
    m1 = jnp.max(logits, axis=1, keepdims=True)
    e1 = jnp.min(jnp.where(logits == m1, lane, E), axis=1, keepdims=True)
    masked = jnp.where(lane == e1, NEG_INF, logits)
    m2 = jnp.max(masked, axis=1, keepdims=True)
    e2 = jnp.min(jnp.where(masked == m2, lane, E), axis=1, keepdims=True)
    z2 = jnp.exp(m2 - m1)
    g1 = 1.0 / (1.0 + z2)
    g2 = z2 / (1.0 + z2)
    g1_ref[...] = g1
    g2_ref[...] = g2
    e1_s[...] = e1
    e2_s[...] = e2

    onehot1 = (lane == e1).astype(jnp.float32)
    onehot2 = (lane == e2).astype(jnp.float32)
    count1 = jnp.sum(onehot1, axis=0, keepdims=True)
    count2 = jnp.sum(onehot2, axis=0, keepdims=True)
    counts = count1 + count2

    gates = jnp.where(lane == e1, g1, 0.0) + jnp.where(lane == e2, g2, 0.0)
    importance = jnp.sum(gates, axis=0, keepdims=True)
    load = jnp.sum((gates > 0.0).astype(jnp.float32), axis=0, keepdims=True)

    def cv2(v):
        m = jnp.sum(v) / E
        var = jnp.sum((v - m) ** 2) / (E - 1)
        return var / (m * m + 1e-10)

    loss_ref[0, 0] = (cv2(importance) + cv2(load)) * coef_ref[0, 0]

    r64 = lax.broadcasted_iota(jnp.int32, (E, E), 0)
    c64 = lax.broadcasted_iota(jnp.int32, (E, E), 1)
    excl64 = (r64 < c64).astype(jnp.float32)
    offs = jnp.dot(counts, excl64, preferred_element_type=jnp.float32)
    base1 = offs
    base2 = offs + count1

    incl64 = (r64 <= c64).astype(jnp.float32)
    off_incl = jnp.dot(counts, incl64, preferred_element_type=jnp.float32)
    off_excl = offs
    cnt_pos = counts > 0.0
    ft = jnp.floor(off_excl * (1.0 / TM))
    lt = jnp.where(cnt_pos, jnp.floor((off_incl - 1.0) * (1.0 / TM)), ft)
    items = jnp.where(cnt_pos, lt - ft + 1.0, 0.0)
    sitem_excl = jnp.dot(items, excl64, preferred_element_type=jnp.float32)
    sitem_incl = sitem_excl + items
    total = jnp.sum(items)
    jcol = lax.broadcasted_iota(
        jnp.int32, (NSTEPS, 1), 0).astype(jnp.float32)
    e_of = jnp.sum((sitem_incl <= jcol).astype(jnp.float32),
                   axis=1, keepdims=True)
    e_ofc = jnp.minimum(e_of, float(E - 1))
    lane_s = lax.broadcasted_iota(jnp.int32, (NSTEPS, E), 1)
    onehot_e = (lane_s == e_ofc.astype(jnp.int32)).astype(jnp.float32)
    ft_j = jnp.sum(onehot_e * ft, axis=1, keepdims=True)
    se_j = jnp.sum(onehot_e * sitem_excl, axis=1, keepdims=True)
    gs_j = jnp.sum(onehot_e * off_excl, axis=1, keepdims=True)
    ge_j = jnp.sum(onehot_e * off_incl, axis=1, keepdims=True)
    valid = jcol < total
    t_of = jnp.where(valid, ft_j + (jcol - se_j), float(NT - 1))
    e_pad = jnp.sum(jnp.where(jcol == total - 1.0, e_ofc, 0.0))
    e_fin = jnp.where(valid, e_ofc, e_pad)
    gs_f = jnp.where(valid, gs_j, 0.0)
    ge_f = jnp.where(valid, ge_j, 0.0)
    row0 = t_of * TM
    first = valid & (gs_f <= row0) & (row0 < ge_f)
    steps = jnp.concatenate(
        [t_of, e_fin, gs_f, ge_f,
         valid.astype(jnp.float32), first.astype(jnp.float32),
         jnp.zeros((NSTEPS, 2), jnp.float32)], axis=1)
    steps_ref[...] = steps.astype(jnp.int32)

    rblk = lax.broadcasted_iota(jnp.int32, (BLK, BLK), 0)
    cblk = lax.broadcasted_iota(jnp.int32, (BLK, BLK), 1)
    tri = (rblk > cblk).astype(jnp.float32)
    lane_b = lax.broadcasted_iota(jnp.int32, (BLK, E), 1)

    def body(b, carry):
        run1, run2 = carry
        e1b = e1_s[pl.ds(b * BLK, BLK), :]
        e2b = e2_s[pl.ds(b * BLK, BLK), :]
        oh1 = (lane_b == e1b).astype(jnp.float32)
        oh2 = (lane_b == e2b).astype(jnp.float32)
        pref1 = jnp.dot(tri, oh1, preferred_element_type=jnp.float32) + run1
        pref2 = jnp.dot(tri, oh2, preferred_element_type=jnp.float32) + run2
        p0 = (jnp.sum(pref1 * oh1, axis=1, keepdims=True)
              + jnp.sum(oh1 * base1, axis=1, keepdims=True))
        p1 = (jnp.sum(pref2 * oh2, axis=1, keepdims=True)
              + jnp.sum(oh2 * base2, axis=1, keepdims=True))
        pos0_ref[pl.ds(b * BLK, BLK), :] = p0.astype(jnp.int32)
        pos1_ref[pl.ds(b * BLK, BLK), :] = p1.astype(jnp.int32)
        return (run1 + jnp.sum(oh1, axis=0, keepdims=True),
                run2 + jnp.sum(oh2, axis=0, keepdims=True))

    lax.fori_loop(0, NBLK, body,
                  (jnp.zeros((1, E), jnp.float32),
                   jnp.zeros((1, E), jnp.float32)))


def _router_call(x, coef, w_gate):
    return pl.pallas_call(
        _router_kernel,
        in_specs=[
            pl.BlockSpec(memory_space=pltpu.HBM),
            pl.BlockSpec((D, E), lambda: (0, 0)),
            pl.BlockSpec(memory_space=pltpu.SMEM),
        ],
        out_specs=[
            pl.BlockSpec((N, 1), lambda: (0, 0)),
            pl.BlockSpec((N, 1), lambda: (0, 0)),
            pl.BlockSpec((N, 1), lambda: (0, 0)),
            pl.BlockSpec((N, 1), lambda: (0, 0)),
            pl.BlockSpec((NSTEPS, 8), lambda: (0, 0)),
            pl.BlockSpec(memory_space=pltpu.SMEM),
            pl.BlockSpec((N, D), lambda: (0, 0)),
        ],
        out_shape=[
            jax.ShapeDtypeStruct((N, 1), jnp.int32),
            jax.ShapeDtypeStruct((N, 1), jnp.int32),
            jax.ShapeDtypeStruct((N, 1), jnp.float32),
            jax.ShapeDtypeStruct((N, 1), jnp.float32),
            jax.ShapeDtypeStruct((NSTEPS, 8), jnp.int32),
            jax.ShapeDtypeStruct((1, 1), jnp.float32),
            jax.ShapeDtypeStruct((N, D), jnp.float32),
        ],
        scratch_shapes=[
            pltpu.VMEM((N, 1), jnp.int32),
            pltpu.VMEM((N, 1), jnp.int32),
            pltpu.VMEM((N, D), jnp.float32),
            pltpu.SemaphoreType.DMA,
        ],
    )(x, w_gate, coef)


def _gmm_kernel(steps_ref, xs_ref, W_ref, b_ref, out_ref):
    s = pl.program_id(0)
    valid = steps_ref[s, 4]

    @pl.when(valid == 1)
    def _():
        t = steps_ref[s, 0]
        g_start = steps_ref[s, 2]
        g_end = steps_ref[s, 3]
        first = steps_ref[s, 5]
        rows = t * TM + lax.broadcasted_iota(jnp.int32, (TM, 1), 0)
        in_seg = (rows >= g_start) & (rows < g_end)
        out = jnp.dot(xs_ref[...], W_ref[0],
                      preferred_element_type=jnp.float32) + b_ref[0]
        expo = jnp.exp(out)
        prev = jnp.where(first == 1, jnp.zeros_like(expo), out_ref[...])
        out_ref[...] = jnp.where(in_seg, expo, prev)


def _gmm_call(steps, xs, expert_W, expert_b):
    grid_spec = pltpu.PrefetchScalarGridSpec(
        num_scalar_prefetch=1,
        grid=(NSTEPS,),
        in_specs=[
            pl.BlockSpec((TM, D), lambda s, st: (st[s, 0], 0)),
            pl.BlockSpec((1, D, D), lambda s, st: (st[s, 1], 0, 0)),
            pl.BlockSpec((1, 1, D), lambda s, st: (st[s, 1], 0, 0)),
        ],
        out_specs=pl.BlockSpec((TM, D), lambda s, st: (st[s, 0], 0)),
    )
    return pl.pallas_call(
        _gmm_kernel,
        grid_spec=grid_spec,
        out_shape=jax.ShapeDtypeStruct((NSLOT, D), jnp.float32),
        compiler_params=pltpu.CompilerParams(
            dimension_semantics=("arbitrary",)),
    )(steps, xs, expert_W, expert_b.reshape(E, 1, D))


@functools.cache
def _sc_kernels():
    mesh = plsc.VectorSubcoreMesh(core_axis_name="c", subcore_axis_name="s")

    @functools.partial(
        pl.kernel,
        mesh=mesh,
        out_type=jax.ShapeDtypeStruct((NSLOT, D), jnp.float32),
        scratch_types=[
            pltpu.VMEM((TPW,), jnp.int32),
            pltpu.VMEM((TPW,), jnp.int32),
            pltpu.VMEM((TPW, D), jnp.float32),
            pltpu.SemaphoreType.DMA,
            pltpu.SemaphoreType.DMA,
        ],
    )
    def dispatch_sc(flat_hbm, pos0_hbm, pos1_hbm, xs_hbm,
                    idx0_v, idx1_v, rows_v, sem0, sem1):
        wid = lax.axis_index("s") * 2 + lax.axis_index("c")
        base = wid * TPW
        pltpu.sync_copy(pos0_hbm.at[pl.ds(base, TPW)], idx0_v)
        pltpu.sync_copy(pos1_hbm.at[pl.ds(base, TPW)], idx1_v)
        pltpu.sync_copy(flat_hbm.at[pl.ds(base, TPW)], rows_v)
        c0 = pltpu.async_copy(rows_v, xs_hbm.at[idx0_v], sem0)
        c1 = pltpu.async_copy(rows_v, xs_hbm.at[idx1_v], sem1)
        c0.wait()
        c1.wait()

    @functools.partial(
        pl.kernel,
        mesh=mesh,
        out_type=[
            jax.ShapeDtypeStruct((N, D), jnp.float32),
            jax.ShapeDtypeStruct((N, D), jnp.float32),
        ],
        scratch_types=[
            pltpu.VMEM((TPW,), jnp.int32),
            pltpu.VMEM((TPW,), jnp.int32),
            pltpu.VMEM((TPW, D), jnp.float32),
            pltpu.VMEM((TPW, D), jnp.float32),
            pltpu.SemaphoreType.DMA,
            pltpu.SemaphoreType.DMA,
        ],
    )
    def combine_sc(expo_hbm, pos0_hbm, pos1_hbm, c0_hbm, c1_hbm,
                   idx0_v, idx1_v, rows0_v, rows1_v, sem0, sem1):
        wid = lax.axis_index("s") * 2 + lax.axis_index("c")
        base = wid * TPW
        pltpu.sync_copy(pos0_hbm.at[pl.ds(base, TPW)], idx0_v)
        pltpu.sync_copy(pos1_hbm.at[pl.ds(base, TPW)], idx1_v)
        g0 = pltpu.async_copy(expo_hbm.at[idx0_v], rows0_v, sem0)
        g1 = pltpu.async_copy(expo_hbm.at[idx1_v], rows1_v, sem1)
        g0.wait()
        g1.wait()
        pltpu.sync_copy(rows0_v, c0_hbm.at[pl.ds(base, TPW)])
        pltpu.sync_copy(rows1_v, c1_hbm.at[pl.ds(base, TPW)])

    return dispatch_sc, combine_sc


def _final_kernel(c0_ref, c1_ref, g1_ref, g2_ref, y_ref):
    s = g1_ref[...] * c0_ref[...] + g2_ref[...] * c1_ref[...]
    y_ref[...] = jnp.log(jnp.where(s == 0.0, EPS, s))


def _final_call(c0, c1, g1, g2):
    nb = 8
    blk = N // nb
    return pl.pallas_call(
        _final_kernel,
        grid=(nb,),
        in_specs=[
            pl.BlockSpec((blk, D), lambda i: (i, 0)),
            pl.BlockSpec((blk, D), lambda i: (i, 0)),
            pl.BlockSpec((blk, 1), lambda i: (i, 0)),
            pl.BlockSpec((blk, 1), lambda i: (i, 0)),
        ],
        out_specs=pl.BlockSpec((blk, D), lambda i: (i, 0)),
        out_shape=jax.ShapeDtypeStruct((N, D), jnp.float32),
    )(c0, c1, g1, g2)


def kernel(x, loss_coef, w_gate, expert_W, expert_b):
    coef = loss_coef.reshape(1, 1)
    pos0, pos1, g1, g2, steps, loss, flat_lin = _router_call(
        x, coef, w_gate)
    p0 = pos0.reshape(N)
    p1 = pos1.reshape(N)
    dispatch_sc, combine_sc = _sc_kernels()
    xs = dispatch_sc(flat_lin, p0, p1)
    expo = _gmm_call(steps, xs, expert_W, expert_b)
    c0, c1 = combine_sc(expo, p0, p1)
    y = _final_call(c0, c1, g1, g2)
    return y, loss[0, 0]


# speedup vs baseline: 1.0616x; 1.0206x over previous
import functools

import jax
import jax.numpy as jnp
from jax import lax
from jax.experimental import pallas as pl
from jax.experimental.pallas import tpu as pltpu
from jax.experimental.pallas import tpu_sc as plsc

N, D, E = 2048, 768, 64
NSLOT = 2 * N
TM = 256
NT = NSLOT // TM
NSTEPS = NT + E - 1
BLK = 256
NBLK = N // BLK
NW = 32
TPW = N // NW
SPW = NSLOT // NW
EPS = 2.220446049250313e-16
NEG_INF = float("-inf")


def _router_kernel(x_hbm, wg_ref, coef_ref,
                   pos0_ref, pos1_ref, g1_ref, g2_ref, steps_ref, loss_ref,
                   flat_lin_ref, e1_s, e2_s, flat_s, dma_sem):
    cp = pltpu.make_async_copy(x_hbm.at[:, 0, :], flat_s, dma_sem)
    cp.start()
    cp.wait()
    flat = flat_s[...]
    flat_lin_ref[...] = flat
    logits = lax.dot_general(flat, wg_ref[...], (((1,), (1,)), ((), ())),
                             preferred_element_type=jnp.float32)
    lane = lax.broadcasted_iota(jnp.int32, (N, E), 1)
    m1 = jnp.max(logits, axis=1, keepdims=True)
    e1 = jnp.min(jnp.where(logits == m1, lane, E), axis=1, keepdims=True)
    masked = jnp.where(lane == e1, NEG_INF, logits)
    m2 = jnp.max(masked, axis=1, keepdims=True)
    e2 = jnp.min(jnp.where(masked == m2, lane, E), axis=1, keepdims=True)
    z2 = jnp.exp(m2 - m1)
    g1 = 1.0 / (1.0 + z2)
    g2 = z2 / (1.0 + z2)
    g1_ref[...] = g1
    g2_ref[...] = g2
    e1_s[...] = e1
    e2_s[...] = e2

    onehot1 = (lane == e1).astype(jnp.float32)
    onehot2 = (lane == e2).astype(jnp.float32)
    count1 = jnp.sum(onehot1, axis=0, keepdims=True)
    count2 = jnp.sum(onehot2, axis=0, keepdims=True)
    counts = count1 + count2

    gates = jnp.where(lane == e1, g1, 0.0) + jnp.where(lane == e2, g2, 0.0)
    importance = jnp.sum(gates, axis=0, keepdims=True)
    load = jnp.sum((gates > 0.0).astype(jnp.float32), axis=0, keepdims=True)

    def cv2(v):
        m = jnp.sum(v) / E
        var = jnp.sum((v - m) ** 2) / (E - 1)
        return var / (m * m + 1e-10)

    loss_ref[0, 0] = (cv2(importance) + cv2(load)) * coef_ref[0, 0]

    r64 = lax.broadcasted_iota(jnp.int32, (E, E), 0)
    c64 = lax.broadcasted_iota(jnp.int32, (E, E), 1)
    excl64 = (r64 < c64).astype(jnp.float32)
    offs = jnp.dot(counts, excl64, preferred_element_type=jnp.float32)
    base1 = offs
    base2 = offs + count1

    incl64 = (r64 <= c64).astype(jnp.float32)
    off_incl = jnp.dot(counts, incl64, preferred_element_type=jnp.float32)
    off_excl = offs
    cnt_pos = counts > 0.0
    ft = jnp.floor(off_excl * (1.0 / TM))
    lt = jnp.where(cnt_pos, jnp.floor((off_incl - 1.0) * (1.0 / TM)), ft)
    items = jnp.where(cnt_pos, lt - ft + 1.0, 0.0)
    sitem_excl = jnp.dot(items, excl64, preferred_element_type=jnp.float32)
    sitem_incl = sitem_excl + items
    total = jnp.sum(items)
    jcol = lax.broadcasted_iota(
        jnp.int32, (NSTEPS, 1), 0).astype(jnp.float32)
    e_of = jnp.sum((sitem_incl <= jcol).astype(jnp.float32),
                   axis=1, keepdims=True)
    e_ofc = jnp.minimum(e_of, float(E - 1))
    lane_s = lax.broadcasted_iota(jnp.int32, (NSTEPS, E), 1)
    onehot_e = (lane_s == e_ofc.astype(jnp.int32)).astype(jnp.float32)
    ft_j = jnp.sum(onehot_e * ft, axis=1, keepdims=True)
    se_j = jnp.sum(onehot_e * sitem_excl, axis=1, keepdims=True)
    gs_j = jnp.sum(onehot_e * off_excl, axis=1, keepdims=True)
    ge_j = jnp.sum(onehot_e * off_incl, axis=1, keepdims=True)
    valid = jcol < total
    t_of = jnp.where(valid, ft_j + (jcol - se_j), float(NT - 1))
    e_pad = jnp.sum(jnp.where(jcol == total - 1.0, e_ofc, 0.0))
    e_fin = jnp.where(valid, e_ofc, e_pad)
    gs_f = jnp.where(valid, gs_j, 0.0)
    ge_f = jnp.where(valid, ge_j, 0.0)
    row0 = t_of * TM
    first = valid & (gs_f <= row0) & (row0 < ge_f)
    steps = jnp.concatenate(
        [t_of, e_fin, gs_f, ge_f,
         valid.astype(jnp.float32), first.astype(jnp.float32),
         jnp.zeros((NSTEPS, 2), jnp.float32)], axis=1)
    steps_ref[...] = steps.astype(jnp.int32)

    rblk = lax.broadcasted_iota(jnp.int32, (BLK, BLK), 0)
    cblk = lax.broadcasted_iota(jnp.int32, (BLK, BLK), 1)
    tri = (rblk > cblk).astype(jnp.float32)
    eye = (rblk == cblk).astype(jnp.float32)
    lane_b = lax.broadcasted_iota(jnp.int32, (BLK, E), 1)

    def body(b, carry):
        run1, run2 = carry
        e1b = e1_s[pl.ds(b * BLK, BLK), :]
        e2b = e2_s[pl.ds(b * BLK, BLK), :]
        oh1 = (lane_b == e1b).astype(jnp.float32)
        oh2 = (lane_b == e2b).astype(jnp.float32)
        pref1 = jnp.dot(tri, oh1, preferred_element_type=jnp.float32) + run1
        pref2 = jnp.dot(tri, oh2, preferred_element_type=jnp.float32) + run2
        p0 = (jnp.sum(pref1 * oh1, axis=1, keepdims=True)
              + jnp.sum(oh1 * base1, axis=1, keepdims=True))
        p1 = (jnp.sum(pref2 * oh2, axis=1, keepdims=True)
              + jnp.sum(oh2 * base2, axis=1, keepdims=True))
        tr = (((0,), (0,)), ((), ()))
        p0r = lax.dot_general(p0, eye, tr,
                              preferred_element_type=jnp.float32)
        p1r = lax.dot_general(p1, eye, tr,
                              preferred_element_type=jnp.float32)
        pos0_ref[pl.ds(b * BLK, BLK)] = jnp.reshape(
            p0r.astype(jnp.int32), (BLK,))
        pos1_ref[pl.ds(b * BLK, BLK)] = jnp.reshape(
            p1r.astype(jnp.int32), (BLK,))
        return (run1 + jnp.sum(oh1, axis=0, keepdims=True),
                run2 + jnp.sum(oh2, axis=0, keepdims=True))

    lax.fori_loop(0, NBLK, body,
                  (jnp.zeros((1, E), jnp.float32),
                   jnp.zeros((1, E), jnp.float32)))


def _router_call(x, coef, w_gate):
    return pl.pallas_call(
        _router_kernel,
        in_specs=[
            pl.BlockSpec(memory_space=pltpu.HBM),
            pl.BlockSpec((E, D), lambda: (0, 0)),
            pl.BlockSpec(memory_space=pltpu.SMEM),
        ],
        out_specs=[
            pl.BlockSpec((N,), lambda: (0,)),
            pl.BlockSpec((N,), lambda: (0,)),
            pl.BlockSpec((N, 1), lambda: (0, 0)),
            pl.BlockSpec((N, 1), lambda: (0, 0)),
            pl.BlockSpec((NSTEPS, 8), lambda: (0, 0)),
            pl.BlockSpec(memory_space=pltpu.SMEM),
            pl.BlockSpec((N, D), lambda: (0, 0)),
        ],
        out_shape=[
            jax.ShapeDtypeStruct((N,), jnp.int32),
            jax.ShapeDtypeStruct((N,), jnp.int32),
            jax.ShapeDtypeStruct((N, 1), jnp.float32),
            jax.ShapeDtypeStruct((N, 1), jnp.float32),
            jax.ShapeDtypeStruct((NSTEPS, 8), jnp.int32),
            jax.ShapeDtypeStruct((1, 1), jnp.float32),
            jax.ShapeDtypeStruct((N, D), jnp.float32),
        ],
        scratch_shapes=[
            pltpu.VMEM((N, 1), jnp.int32),
            pltpu.VMEM((N, 1), jnp.int32),
            pltpu.VMEM((N, D), jnp.float32),
            pltpu.SemaphoreType.DMA,
        ],
    )(x, w_gate, coef)


def _gmm_kernel(steps_ref, xs_ref, W_ref, b_ref, out_ref):
    s = pl.program_id(0)
    valid = steps_ref[s, 4]

    @pl.when(valid == 1)
    def _():
        t = steps_ref[s, 0]
        g_start = steps_ref[s, 2]
        g_end = steps_ref[s, 3]
        first = steps_ref[s, 5]
        rows = t * TM + lax.broadcasted_iota(jnp.int32, (TM, 1), 0)
        in_seg = (rows >= g_start) & (rows < g_end)
        out = jnp.dot(xs_ref[...], W_ref[0],
                      preferred_element_type=jnp.float32) + b_ref[0]
        expo = jnp.exp(out)
        prev = jnp.where(first == 1, jnp.zeros_like(expo), out_ref[...])
        out_ref[...] = jnp.where(in_seg, expo, prev)


def _gmm_call(steps, xs, expert_W, expert_b):
    grid_spec = pltpu.PrefetchScalarGridSpec(
        num_scalar_prefetch=1,
        grid=(NSTEPS,),
        in_specs=[
            pl.BlockSpec((TM, D), lambda s, st: (st[s, 0], 0)),
            pl.BlockSpec((1, D, D), lambda s, st: (st[s, 1], 0, 0)),
            pl.BlockSpec((1, 1, D), lambda s, st: (st[s, 1], 0, 0)),
        ],
        out_specs=pl.BlockSpec((TM, D), lambda s, st: (st[s, 0], 0)),
    )
    return pl.pallas_call(
        _gmm_kernel,
        grid_spec=grid_spec,
        out_shape=jax.ShapeDtypeStruct((NSLOT, D), jnp.float32),
        compiler_params=pltpu.CompilerParams(
            dimension_semantics=("arbitrary",)),
    )(steps, xs, expert_W, expert_b.reshape(E, 1, D))


@functools.cache
def _sc_kernels():
    mesh = plsc.VectorSubcoreMesh(core_axis_name="c", subcore_axis_name="s")

    @functools.partial(
        pl.kernel,
        mesh=mesh,
        out_type=jax.ShapeDtypeStruct((NSLOT, D), jnp.float32),
        scratch_types=[
            pltpu.VMEM((TPW,), jnp.int32),
            pltpu.VMEM((TPW,), jnp.int32),
            pltpu.VMEM((TPW, D), jnp.float32),
            pltpu.SemaphoreType.DMA,
            pltpu.SemaphoreType.DMA,
        ],
    )
    def dispatch_sc(flat_hbm, pos0_hbm, pos1_hbm, xs_hbm,
                    idx0_v, idx1_v, rows_v, sem0, sem1):
        wid = lax.axis_index("s") * 2 + lax.axis_index("c")
        base = wid * TPW
        pltpu.sync_copy(pos0_hbm.at[pl.ds(base, TPW)], idx0_v)
        pltpu.sync_copy(pos1_hbm.at[pl.ds(base, TPW)], idx1_v)
        pltpu.sync_copy(flat_hbm.at[pl.ds(base, TPW)], rows_v)
        c0 = pltpu.async_copy(rows_v, xs_hbm.at[idx0_v], sem0)
        c1 = pltpu.async_copy(rows_v, xs_hbm.at[idx1_v], sem1)
        c0.wait()
        c1.wait()

    @functools.partial(
        pl.kernel,
        mesh=mesh,
        out_type=[
            jax.ShapeDtypeStruct((N, D), jnp.float32),
            jax.ShapeDtypeStruct((N, D), jnp.float32),
        ],
        scratch_types=[
            pltpu.VMEM((TPW,), jnp.int32),
            pltpu.VMEM((TPW,), jnp.int32),
            pltpu.VMEM((TPW, D), jnp.float32),
            pltpu.VMEM((TPW, D), jnp.float32),
            pltpu.SemaphoreType.DMA,
            pltpu.SemaphoreType.DMA,
        ],
    )
    def combine_sc(expo_hbm, pos0_hbm, pos1_hbm, c0_hbm, c1_hbm,
                   idx0_v, idx1_v, rows0_v, rows1_v, sem0, sem1):
        wid = lax.axis_index("s") * 2 + lax.axis_index("c")
        base = wid * TPW
        pltpu.sync_copy(pos0_hbm.at[pl.ds(base, TPW)], idx0_v)
        pltpu.sync_copy(pos1_hbm.at[pl.ds(base, TPW)], idx1_v)
        g0 = pltpu.async_copy(expo_hbm.at[idx0_v], rows0_v, sem0)
        g1 = pltpu.async_copy(expo_hbm.at[idx1_v], rows1_v, sem1)
        g0.wait()
        g1.wait()
        pltpu.sync_copy(rows0_v, c0_hbm.at[pl.ds(base, TPW)])
        pltpu.sync_copy(rows1_v, c1_hbm.at[pl.ds(base, TPW)])

    return dispatch_sc, combine_sc


def _final_kernel(c0_ref, c1_ref, g1_ref, g2_ref, y_ref):
    s = g1_ref[...] * c0_ref[...] + g2_ref[...] * c1_ref[...]
    y_ref[...] = jnp.log(jnp.where(s == 0.0, EPS, s))


def _final_call(c0, c1, g1, g2):
    nb = 8
    blk = N // nb
    return pl.pallas_call(
        _final_kernel,
        grid=(nb,),
        in_specs=[
            pl.BlockSpec((blk, D), lambda i: (i, 0)),
            pl.BlockSpec((blk, D), lambda i: (i, 0)),
            pl.BlockSpec((blk, 1), lambda i: (i, 0)),
            pl.BlockSpec((blk, 1), lambda i: (i, 0)),
        ],
        out_specs=pl.BlockSpec((blk, D), lambda i: (i, 0)),
        out_shape=jax.ShapeDtypeStruct((N, D), jnp.float32),
    )(c0, c1, g1, g2)


def kernel(x, loss_coef, w_gate, expert_W, expert_b):
    coef = loss_coef.reshape(1, 1)
    pos0, pos1, g1, g2, steps, loss, flat_lin = _router_call(
        x, coef, w_gate.T)
    dispatch_sc, combine_sc = _sc_kernels()
    xs = dispatch_sc(flat_lin, pos0, pos1)
    expo = _gmm_call(steps, xs, expert_W, expert_b)
    c0, c1 = combine_sc(expo, pos0, pos1)
    y = _final_call(c0, c1, g1, g2)
    return y, loss[0, 0]


# speedup vs baseline: 1.0698x; 1.0078x over previous
import functools

import jax
import jax.numpy as jnp
from jax import lax
from jax.experimental import pallas as pl
from jax.experimental.pallas import tpu as pltpu
from jax.experimental.pallas import tpu_sc as plsc

N, D, E = 2048, 768, 64
NSLOT = 2 * N
TM = 256
NT = NSLOT // TM
NSTEPS = NT + E - 1
BLK = 256
NBLK = N // BLK
NW = 32
TPW = N // NW
SPW = NSLOT // NW
EPS = 2.220446049250313e-16
NEG_INF = float("-inf")


def _router_kernel(x_hbm, wg_ref, coef_ref,
                   pos0_ref, pos1_ref, g1_ref, g2_ref, steps_ref, loss_ref,
                   e1_s, e2_s, flat_s, dma_sem):
    cp = pltpu.make_async_copy(x_hbm.at[:, 0, :], flat_s, dma_sem)
    cp.start()
    cp.wait()
    flat = flat_s[...]
    logits = lax.dot_general(flat, wg_ref[...], (((1,), (1,)), ((), ())),
                             preferred_element_type=jnp.float32)
    lane = lax.broadcasted_iota(jnp.int32, (N, E), 1)
    m1 = jnp.max(logits, axis=1, keepdims=True)
    e1 = jnp.min(jnp.where(logits == m1, lane, E), axis=1, keepdims=True)
    masked = jnp.where(lane == e1, NEG_INF, logits)
    m2 = jnp.max(masked, axis=1, keepdims=True)
    e2 = jnp.min(jnp.where(masked == m2, lane, E), axis=1, keepdims=True)
    z2 = jnp.exp(m2 - m1)
    g1 = 1.0 / (1.0 + z2)
    g2 = z2 / (1.0 + z2)
    g1_ref[...] = g1
    g2_ref[...] = g2
    e1_s[...] = e1
    e2_s[...] = e2

    onehot1 = (lane == e1).astype(jnp.float32)
    onehot2 = (lane == e2).astype(jnp.float32)
    count1 = jnp.sum(onehot1, axis=0, keepdims=True)
    count2 = jnp.sum(onehot2, axis=0, keepdims=True)
    counts = count1 + count2

    gates = jnp.where(lane == e1, g1, 0.0) + jnp.where(lane == e2, g2, 0.0)
    importance = jnp.sum(gates, axis=0, keepdims=True)
    load = jnp.sum((gates > 0.0).astype(jnp.float32), axis=0, keepdims=True)

    def cv2(v):
        m = jnp.sum(v) / E
        var = jnp.sum((v - m) ** 2) / (E - 1)
        return var / (m * m + 1e-10)

    loss_ref[0, 0] = (cv2(importance) + cv2(load)) * coef_ref[0, 0]

    r64 = lax.broadcasted_iota(jnp.int32, (E, E), 0)
    c64 = lax.broadcasted_iota(jnp.int32, (E, E), 1)
    excl64 = (r64 < c64).astype(jnp.float32)
    offs = jnp.dot(counts, excl64, preferred_element_type=jnp.float32)
    base1 = offs
    base2 = offs + count1

    incl64 = (r64 <= c64).astype(jnp.float32)
    off_incl = jnp.dot(counts, incl64, preferred_element_type=jnp.float32)
    off_excl = offs
    cnt_pos = counts > 0.0
    ft = jnp.floor(off_excl * (1.0 / TM))
    lt = jnp.where(cnt_pos, jnp.floor((off_incl - 1.0) * (1.0 / TM)), ft)
    items = jnp.where(cnt_pos, lt - ft + 1.0, 0.0)
    sitem_excl = jnp.dot(items, excl64, preferred_element_type=jnp.float32)
    sitem_incl = sitem_excl + items
    total = jnp.sum(items)
    jcol = lax.broadcasted_iota(
        jnp.int32, (NSTEPS, 1), 0).astype(jnp.float32)
    e_of = jnp.sum((sitem_incl <= jcol).astype(jnp.float32),
                   axis=1, keepdims=True)
    e_ofc = jnp.minimum(e_of, float(E - 1))
    lane_s = lax.broadcasted_iota(jnp.int32, (NSTEPS, E), 1)
    onehot_e = (lane_s == e_ofc.astype(jnp.int32)).astype(jnp.float32)
    ft_j = jnp.sum(onehot_e * ft, axis=1, keepdims=True)
    se_j = jnp.sum(onehot_e * sitem_excl, axis=1, keepdims=True)
    gs_j = jnp.sum(onehot_e * off_excl, axis=1, keepdims=True)
    ge_j = jnp.sum(onehot_e * off_incl, axis=1, keepdims=True)
    valid = jcol < total
    t_of = jnp.where(valid, ft_j + (jcol - se_j), float(NT - 1))
    e_pad = jnp.sum(jnp.where(jcol == total - 1.0, e_ofc, 0.0))
    e_fin = jnp.where(valid, e_ofc, e_pad)
    gs_f = jnp.where(valid, gs_j, 0.0)
    ge_f = jnp.where(valid, ge_j, 0.0)
    row0 = t_of * TM
    first = valid & (gs_f <= row0) & (row0 < ge_f)
    steps = jnp.concatenate(
        [t_of, e_fin, gs_f, ge_f,
         valid.astype(jnp.float32), first.astype(jnp.float32),
         jnp.zeros((NSTEPS, 2), jnp.float32)], axis=1)
    steps_ref[...] = steps.astype(jnp.int32)

    rblk = lax.broadcasted_iota(jnp.int32, (BLK, BLK), 0)
    cblk = lax.broadcasted_iota(jnp.int32, (BLK, BLK), 1)
    tri = (rblk > cblk).astype(jnp.float32)
    eye = (rblk == cblk).astype(jnp.float32)
    lane_b = lax.broadcasted_iota(jnp.int32, (BLK, E), 1)

    def body(b, carry):
        run1, run2 = carry
        e1b = e1_s[pl.ds(b * BLK, BLK), :]
        e2b = e2_s[pl.ds(b * BLK, BLK), :]
        oh1 = (lane_b == e1b).astype(jnp.float32)
        oh2 = (lane_b == e2b).astype(jnp.float32)
        pref1 = jnp.dot(tri, oh1, preferred_element_type=jnp.float32) + run1
        pref2 = jnp.dot(tri, oh2, preferred_element_type=jnp.float32) + run2
        p0 = (jnp.sum(pref1 * oh1, axis=1, keepdims=True)
              + jnp.sum(oh1 * base1, axis=1, keepdims=True))
        p1 = (jnp.sum(pref2 * oh2, axis=1, keepdims=True)
              + jnp.sum(oh2 * base2, axis=1, keepdims=True))
        tr = (((0,), (0,)), ((), ()))
        p0r = lax.dot_general(p0, eye, tr,
                              preferred_element_type=jnp.float32)
        p1r = lax.dot_general(p1, eye, tr,
                              preferred_element_type=jnp.float32)
        pos0_ref[pl.ds(b * BLK, BLK)] = jnp.reshape(
            p0r.astype(jnp.int32), (BLK,))
        pos1_ref[pl.ds(b * BLK, BLK)] = jnp.reshape(
            p1r.astype(jnp.int32), (BLK,))
        return (run1 + jnp.sum(oh1, axis=0, keepdims=True),
                run2 + jnp.sum(oh2, axis=0, keepdims=True))

    lax.fori_loop(0, NBLK, body,
                  (jnp.zeros((1, E), jnp.float32),
                   jnp.zeros((1, E), jnp.float32)))


def _router_call(x, coef, w_gate):
    return pl.pallas_call(
        _router_kernel,
        in_specs=[
            pl.BlockSpec(memory_space=pltpu.HBM),
            pl.BlockSpec((E, D), lambda: (0, 0)),
            pl.BlockSpec(memory_space=pltpu.SMEM),
        ],
        out_specs=[
            pl.BlockSpec((N,), lambda: (0,)),
            pl.BlockSpec((N,), lambda: (0,)),
            pl.BlockSpec((N, 1), lambda: (0, 0)),
            pl.BlockSpec((N, 1), lambda: (0, 0)),
            pl.BlockSpec((NSTEPS, 8), lambda: (0, 0)),
            pl.BlockSpec(memory_space=pltpu.SMEM),
        ],
        out_shape=[
            jax.ShapeDtypeStruct((N,), jnp.int32),
            jax.ShapeDtypeStruct((N,), jnp.int32),
            jax.ShapeDtypeStruct((N, 1), jnp.float32),
            jax.ShapeDtypeStruct((N, 1), jnp.float32),
            jax.ShapeDtypeStruct((NSTEPS, 8), jnp.int32),
            jax.ShapeDtypeStruct((1, 1), jnp.float32),
        ],
        scratch_shapes=[
            pltpu.VMEM((N, 1), jnp.int32),
            pltpu.VMEM((N, 1), jnp.int32),
            pltpu.VMEM((N, D), jnp.float32),
            pltpu.SemaphoreType.DMA,
        ],
    )(x, w_gate, coef)


def _gmm_kernel(steps_ref, xs_ref, W_ref, b_ref, out_ref):
    s = pl.program_id(0)
    valid = steps_ref[s, 4]

    @pl.when(valid == 1)
    def _():
        t = steps_ref[s, 0]
        g_start = steps_ref[s, 2]
        g_end = steps_ref[s, 3]
        first = steps_ref[s, 5]
        rows = t * TM + lax.broadcasted_iota(jnp.int32, (TM, 1), 0)
        in_seg = (rows >= g_start) & (rows < g_end)
        out = jnp.dot(xs_ref[...], W_ref[0],
                      preferred_element_type=jnp.float32) + b_ref[0]
        expo = jnp.exp(out)
        prev = jnp.where(first == 1, jnp.zeros_like(expo), out_ref[...])
        out_ref[...] = jnp.where(in_seg, expo, prev)


def _gmm_call(steps, xs, expert_W, expert_b):
    grid_spec = pltpu.PrefetchScalarGridSpec(
        num_scalar_prefetch=1,
        grid=(NSTEPS,),
        in_specs=[
            pl.BlockSpec((TM, D), lambda s, st: (st[s, 0], 0)),
            pl.BlockSpec((1, D, D), lambda s, st: (st[s, 1], 0, 0)),
            pl.BlockSpec((1, 1, D), lambda s, st: (st[s, 1], 0, 0)),
        ],
        out_specs=pl.BlockSpec((TM, D), lambda s, st: (st[s, 0], 0)),
    )
    return pl.pallas_call(
        _gmm_kernel,
        grid_spec=grid_spec,
        out_shape=jax.ShapeDtypeStruct((NSLOT, D), jnp.float32),
        compiler_params=pltpu.CompilerParams(
            dimension_semantics=("arbitrary",)),
    )(steps, xs, expert_W, expert_b.reshape(E, 1, D))


@functools.cache
def _sc_kernels():
    mesh = plsc.VectorSubcoreMesh(core_axis_name="c", subcore_axis_name="s")

    @functools.partial(
        pl.kernel,
        mesh=mesh,
        out_type=jax.ShapeDtypeStruct((NSLOT, D), jnp.float32),
        scratch_types=[
            pltpu.VMEM((TPW,), jnp.int32),
            pltpu.VMEM((TPW,), jnp.int32),
            pltpu.VMEM((TPW, D), jnp.float32),
            pltpu.SemaphoreType.DMA,
            pltpu.SemaphoreType.DMA,
        ],
    )
    def dispatch_sc(x_hbm, pos0_hbm, pos1_hbm, xs_hbm,
                    idx0_v, idx1_v, rows_v, sem0, sem1):
        wid = lax.axis_index("s") * 2 + lax.axis_index("c")
        base = wid * TPW
        pltpu.sync_copy(pos0_hbm.at[pl.ds(base, TPW)], idx0_v)
        pltpu.sync_copy(pos1_hbm.at[pl.ds(base, TPW)], idx1_v)
        pltpu.sync_copy(x_hbm.at[pl.ds(base, TPW), 0], rows_v)
        c0 = pltpu.async_copy(rows_v, xs_hbm.at[idx0_v], sem0)
        c1 = pltpu.async_copy(rows_v, xs_hbm.at[idx1_v], sem1)
        c0.wait()
        c1.wait()

    @functools.partial(
        pl.kernel,
        mesh=mesh,
        out_type=[
            jax.ShapeDtypeStruct((N, D), jnp.float32),
            jax.ShapeDtypeStruct((N, D), jnp.float32),
        ],
        scratch_types=[
            pltpu.VMEM((TPW,), jnp.int32),
            pltpu.VMEM((TPW,), jnp.int32),
            pltpu.VMEM((TPW, D), jnp.float32),
            pltpu.VMEM((TPW, D), jnp.float32),
            pltpu.SemaphoreType.DMA,
            pltpu.SemaphoreType.DMA,
        ],
    )
    def combine_sc(expo_hbm, pos0_hbm, pos1_hbm, c0_hbm, c1_hbm,
                   idx0_v, idx1_v, rows0_v, rows1_v, sem0, sem1):
        wid = lax.axis_index("s") * 2 + lax.axis_index("c")
        base = wid * TPW
        pltpu.sync_copy(pos0_hbm.at[pl.ds(base, TPW)], idx0_v)
        pltpu.sync_copy(pos1_hbm.at[pl.ds(base, TPW)], idx1_v)
        g0 = pltpu.async_copy(expo_hbm.at[idx0_v], rows0_v, sem0)
        g1 = pltpu.async_copy(expo_hbm.at[idx1_v], rows1_v, sem1)
        g0.wait()
        g1.wait()
        pltpu.sync_copy(rows0_v, c0_hbm.at[pl.ds(base, TPW)])
        pltpu.sync_copy(rows1_v, c1_hbm.at[pl.ds(base, TPW)])

    return dispatch_sc, combine_sc


def _final_kernel(c0_ref, c1_ref, g1_ref, g2_ref, y_ref):
    s = g1_ref[...] * c0_ref[...] + g2_ref[...] * c1_ref[...]
    y_ref[...] = jnp.log(jnp.where(s == 0.0, EPS, s))


def _final_call(c0, c1, g1, g2):
    nb = 8
    blk = N // nb
    return pl.pallas_call(
        _final_kernel,
        grid=(nb,),
        in_specs=[
            pl.BlockSpec((blk, D), lambda i: (i, 0)),
            pl.BlockSpec((blk, D), lambda i: (i, 0)),
            pl.BlockSpec((blk, 1), lambda i: (i, 0)),
            pl.BlockSpec((blk, 1), lambda i: (i, 0)),
        ],
        out_specs=pl.BlockSpec((blk, D), lambda i: (i, 0)),
        out_shape=jax.ShapeDtypeStruct((N, D), jnp.float32),
    )(c0, c1, g1, g2)


def kernel(x, loss_coef, w_gate, expert_W, expert_b):
    coef = loss_coef.reshape(1, 1)
    pos0, pos1, g1, g2, steps, loss = _router_call(x, coef, w_gate.T)
    dispatch_sc, combine_sc = _sc_kernels()
    xs = dispatch_sc(x, pos0, pos1)
    expo = _gmm_call(steps, xs, expert_W, expert_b)
    c0, c1 = combine_sc(expo, pos0, pos1)
    y = _final_call(c0, c1, g1, g2)
    return y, loss[0, 0]
